# Initial kernel scaffold; baseline (speedup 1.0000x reference)
import jax
import jax.numpy as jnp
from jax.experimental import pallas as pl


def kernel(x, rW, rb, W1, b1, W2, b2):
    raise NotImplementedError("write your pallas kernel here")


# speedup vs baseline: 1.1052x; 1.1052x over previous
import functools

import jax
import jax.numpy as jnp
from jax.experimental import pallas as pl

K = 8


def _router_body(x_ref, rW_ref, rb_ref, mask_ref, load_ref, z_ref, aux_ref, *, nblk, E, T):
    i = pl.program_id(0)
    xb = x_ref[...]
    logits = jnp.dot(xb, rW_ref[...], preferred_element_type=jnp.float32) + rb_ref[...]
    mx = jnp.max(logits, axis=1, keepdims=True)
    ex = jnp.exp(logits - mx)
    se = jnp.sum(ex, axis=1, keepdims=True)
    probs = ex / se
    lse = mx + jnp.log(se)

    iota = jax.lax.broadcasted_iota(jnp.int32, probs.shape, 1)
    work = probs
    acc = jnp.zeros_like(probs)
    for _ in range(K):
        mj = jnp.max(work, axis=1, keepdims=True)
        ismax = work == mj
        sel = jnp.min(jnp.where(ismax, iota, E), axis=1, keepdims=True)
        onehot = iota == sel
        acc = acc + jnp.where(onehot, work, 0.0)
        work = jnp.where(onehot, -jnp.inf, work)
    ssum = jnp.sum(acc, axis=1, keepdims=True)
    maskb = acc / ssum
    mask_ref[...] = maskb

    @pl.when(i == 0)
    def _init():
        load_ref[...] = jnp.zeros_like(load_ref)
        z_ref[...] = jnp.zeros_like(z_ref)

    load_ref[...] += jnp.sum(maskb, axis=0)[None, :]
    z_ref[...] += jnp.reshape(jnp.sum(lse * lse), (1, 1))

    @pl.when(i == nblk - 1)
    def _fin():
        load = load_ref[...] / T
        lb = 0.1 * jnp.sum((load - 1.0 / E) ** 2)
        aux_ref[...] = lb + 0.1 * z_ref[...] / T


def _expert_body(xb_ref, W1_ref, b1_ref, W2_ref, b2_ref, mask_ref, out_ref, *, E):
    e = pl.program_id(0)

    @pl.when(e == 0)
    def _init():
        out_ref[...] = jnp.zeros_like(out_ref)

    xb = xb_ref[...]
    w1 = W1_ref[0].astype(jnp.bfloat16)
    h = jnp.dot(xb, w1, preferred_element_type=jnp.float32) + b1_ref[0]
    h = jnp.maximum(h, 0.0)
    w2 = W2_ref[0].astype(jnp.bfloat16)
    eo = jnp.dot(h.astype(jnp.bfloat16), w2, preferred_element_type=jnp.float32) + b2_ref[0]
    onehot = (jax.lax.broadcasted_iota(jnp.int32, (E, 1), 0) == e).astype(jnp.float32)
    mcol = jnp.dot(mask_ref[...], onehot, preferred_element_type=jnp.float32)
    out_ref[...] += eo * mcol


def kernel(x, rW, rb, W1, b1, W2, b2):
    T, D = x.shape
    E = rW.shape[1]
    H = W1.shape[2]
    C = W2.shape[2]
    TB = 256
    nblk = T // TB

    mask, _load, _z, aux = pl.pallas_call(
        functools.partial(_router_body, nblk=nblk, E=E, T=T),
        grid=(nblk,),
        in_specs=[
            pl.BlockSpec((TB, D), lambda i: (i, 0)),
            pl.BlockSpec((D, E), lambda i: (0, 0)),
            pl.BlockSpec((1, E), lambda i: (0, 0)),
        ],
        out_specs=[
            pl.BlockSpec((TB, E), lambda i: (i, 0)),
            pl.BlockSpec((1, E), lambda i: (0, 0)),
            pl.BlockSpec((1, 1), lambda i: (0, 0)),
            pl.BlockSpec((1, 1), lambda i: (0, 0)),
        ],
        out_shape=[
            jax.ShapeDtypeStruct((T, E), jnp.float32),
            jax.ShapeDtypeStruct((1, E), jnp.float32),
            jax.ShapeDtypeStruct((1, 1), jnp.float32),
            jax.ShapeDtypeStruct((1, 1), jnp.float32),
        ],
    )(x, rW, rb.reshape(1, E))

    xb16 = x.astype(jnp.bfloat16)
    out = pl.pallas_call(
        functools.partial(_expert_body, E=E),
        grid=(E,),
        in_specs=[
            pl.BlockSpec((T, D), lambda e: (0, 0)),
            pl.BlockSpec((1, D, H), lambda e: (e, 0, 0)),
            pl.BlockSpec((1, 1, H), lambda e: (e, 0, 0)),
            pl.BlockSpec((1, H, C), lambda e: (e, 0, 0)),
            pl.BlockSpec((1, 1, C), lambda e: (e, 0, 0)),
            pl.BlockSpec((T, E), lambda e: (0, 0)),
        ],
        out_specs=pl.BlockSpec((T, C), lambda e: (0, 0)),
        out_shape=jax.ShapeDtypeStruct((T, C), jnp.float32),
    )(xb16, W1, b1.reshape(E, 1, H), W2, b2.reshape(E, 1, C), mask)

    return out, aux[0, 0]
